# baseline (speedup 1.0000x reference)
import functools

import jax
import jax.numpy as jnp
from jax import lax
from jax.experimental import pallas as pl
from jax.experimental.pallas import tpu as pltpu
from jax.experimental.pallas import tpu_sc as plsc

R = 128
L = 32768
K = 1024
LANES = 16
NV = L // LANES
CAPL = 512
CAP2 = 2048
HB = 512


def _srl(x, n):
    return lax.shift_right_logical(x, jnp.full(x.shape, n, jnp.int32))


def _sra(x, n):
    return lax.shift_right_arithmetic(x, jnp.full(x.shape, n, jnp.int32))


def _iota():
    return lax.iota(jnp.int32, LANES)


def _splat(v):
    return jnp.full((LANES,), v, jnp.int32)


def _to_ub(f32v):
    b = lax.bitcast_convert_type(f32v, jnp.int32)
    return b ^ (_sra(b, 31) | _splat(-0x80000000))


def _make_kernel():
    info = plsc.get_sparse_core_info()
    nc, ns = info.num_cores, info.num_subcores
    nw = nc * ns
    rpw = R // nw
    mesh = plsc.VectorSubcoreMesh(core_axis_name="c", subcore_axis_name="s",
                                  num_cores=nc, num_subcores=ns)

    @functools.partial(
        pl.kernel,
        mesh=mesh,
        out_type=jax.ShapeDtypeStruct((R, K), jnp.int32),
        compiler_params=pltpu.CompilerParams(needs_layout_passes=False),
        scratch_types=[
            pltpu.VMEM((L,), jnp.float32),
            pltpu.VMEM((LANES * CAPL,), jnp.int32),
            pltpu.VMEM((CAP2,), jnp.int32),
            pltpu.VMEM((CAP2,), jnp.int32),
            pltpu.VMEM((CAP2,), jnp.int32),
            pltpu.VMEM((CAP2,), jnp.int32),
            pltpu.VMEM((LANES * HB,), jnp.int32),
            pltpu.VMEM((HB,), jnp.int32),
            pltpu.VMEM((HB + LANES,), jnp.int32),
            pltpu.VMEM((272,), jnp.int32),
            pltpu.SemaphoreType.DMA,
        ],
    )
    def topk_idx(x_hbm, out_hbm, row_ref, ci, ska, sia, skb, sib,
                 hist, tot, suf, cur, sem):
        cid = lax.axis_index("c")
        sid = lax.axis_index("s")
        wid = sid * nc + cid
        lane = _iota()
        ones = _splat(1)
        zero = _splat(0)
        lane_hb = lane * HB
        lane_cap = lane * CAPL

        def compact8(cu, ms, offs):
            incs = [m.astype(jnp.int32) for m in ms]
            a01 = incs[0] + incs[1]
            a23 = incs[2] + incs[3]
            a45 = incs[4] + incs[5]
            a0123 = a01 + a23
            total = a0123 + a45 + incs[6] + incs[7]
            o = [zero, incs[0], a01, a01 + incs[2], a0123, a0123 + incs[4],
                 a0123 + a45, a0123 + a45 + incs[6]]
            addrs = [cu + ou for ou in o]
            oks = [m & (a < CAPL) for m, a in zip(ms, addrs)]
            for a, ok, of in zip(addrs, oks, offs):
                plsc.store_scatter(ci, [lane_cap + a], lane + of, mask=ok)
            return cu + total

        def reduce_hist(nbins):
            def body(j, _):
                sls = [pl.ds(l * HB + j * LANES, LANES) for l in range(LANES)]
                vs = [hist[sl] for sl in sls]
                for sl in sls:
                    hist[sl] = zero
                while len(vs) > 1:
                    vs = [a + b for a, b in zip(vs[::2], vs[1::2])]
                tot[pl.ds(j * LANES, LANES)] = vs[0]
                return 0
            lax.fori_loop(0, nbins // LANES, body, 0)

        def reduce_hist_remap():
            def pos(j, _):
                sls = [pl.ds(l * HB + j * LANES, LANES) for l in range(LANES)]
                vs = [hist[sl] for sl in sls]
                for sl in sls:
                    hist[sl] = zero
                while len(vs) > 1:
                    vs = [a + b for a, b in zip(vs[::2], vs[1::2])]
                tot[pl.ds(256 + j * LANES, LANES)] = vs[0]
                return 0
            lax.fori_loop(0, 256 // LANES, pos, 0)

            def neg(j, _):
                sls = [pl.ds(l * HB + 256 + j * LANES, LANES)
                       for l in range(LANES)]
                vs = [hist[sl] for sl in sls]
                for sl in sls:
                    hist[sl] = zero
                while len(vs) > 1:
                    vs = [a + b for a, b in zip(vs[::2], vs[1::2])]
                tot[pl.ds(240 - j * LANES, LANES)] = lax.rev(vs[0], (0,))
                return 0
            lax.fori_loop(0, 256 // LANES, neg, 0)

        def suffix_scan(nchunks):
            suf[pl.ds(nchunks * LANES, LANES)] = zero

            def body(i, carry):
                j = nchunks - 1 - i
                sl = pl.ds(j * LANES, LANES)
                v = tot[sl]
                tot[sl] = zero
                c = lax.rev(plsc.cumsum(lax.rev(v, (0,))), (0,)) + carry
                suf[sl] = c
                return plsc.load_gather(suf, [_splat(0) + j * LANES])

            lax.fori_loop(0, nchunks, body, zero)

        def count_ge(nchunks, kneed):
            def body(j, acc):
                m = suf[pl.ds(j * LANES, LANES)] >= kneed
                return acc + plsc.all_reduce_population_count(m)
            return lax.fori_loop(0, nchunks, body, zero)

        def hclear(j, _):
            hist[pl.ds(j * LANES, LANES)] = zero
            return 0
        lax.fori_loop(0, LANES * HB // LANES, hclear, 0)

        def tclear(j, _):
            tot[pl.ds(j * LANES, LANES)] = zero
            return 0
        lax.fori_loop(0, HB // LANES, tclear, 0)

        pltpu.async_copy(x_hbm.at[wid * rpw], row_ref, sem)

        def do_row(r, bspec):
            row = wid * rpw + r
            pltpu.make_async_copy(x_hbm.at[row], row_ref, sem).wait()

            vthresh = jnp.where(bspec >= 256, (bspec - 256) * (1 << 23),
                                _splat(0x7FFFF000))

            def p1(i, cu):
                sls = [pl.ds((i * 8 + u) * LANES, LANES) for u in range(8)]
                offs = [(i * 8 + u) * LANES for u in range(8)]
                bs = [lax.bitcast_convert_type(row_ref[sl], jnp.int32)
                      for sl in sls]
                d1s = [_srl(b, 23) for b in bs]
                ms = [b >= vthresh for b in bs]
                for d1 in d1s:
                    plsc.addupdate_scatter(hist, [lane_hb + d1], ones)
                return compact8(cu, ms, offs)

            cuspec = lax.fori_loop(0, NV // 8, p1, zero)
            reduce_hist_remap()
            suffix_scan(HB // LANES)
            b1 = count_ge(HB // LANES, _splat(K)) - 1
            g1 = plsc.load_gather(suf, [b1 + 1])

            fits = plsc.all_reduce_population_count(cuspec <= CAPL)
            hit = ((b1 >= bspec) & (fits == LANES)
                   & (bspec >= 256)).astype(jnp.int32)
            cur[pl.ds(0, LANES)] = cuspec

            @pl.when(hit[0] == 0)
            def _p2():
                def p2(i, cu):
                    offs = [(i * 8 + u) * LANES for u in range(8)]
                    bs = [lax.bitcast_convert_type(
                        row_ref[pl.ds(o, LANES)], jnp.int32) for o in offs]
                    ss = [_sra(b, 31) | _splat(-0x80000000) for b in bs]
                    ms = [_srl(b ^ sgn, 23) >= b1 for b, sgn in zip(bs, ss)]
                    return compact8(cu, ms, offs)

                cur[pl.ds(0, LANES)] = lax.fori_loop(0, NV // 8, p2, zero)

            wcnt = jnp.minimum(cur[pl.ds(0, LANES)], CAPL)

            NB = 4

            def region(l, body_fn, carry):
                cl = wcnt[l]
                cls = jnp.full((LANES,), cl, jnp.int32)

                def wrap(j, c):
                    poss = [(j * NB + u) * LANES for u in range(NB)]
                    valids = [(lane + p) < cls for p in poss]
                    idxs = [ci[pl.ds(l * CAPL + p, LANES)] & (L - 1)
                            for p in poss]
                    ubs = [_to_ub(plsc.load_gather(row_ref, [ix], mask=v))
                           for ix, v in zip(idxs, valids)]
                    return body_fn(idxs, ubs, valids, c)

                return lax.fori_loop(
                    0, lax.div(cl + NB * LANES - 1, NB * LANES), wrap, carry)

            def whist(idxs, ubs, valids, c):
                ms = [v & (_srl(ub, 23) == b1) for ub, v in zip(ubs, valids)]
                d2s = [lane_hb + (_srl(ub, 15) & 255) for ub in ubs]
                for d2, m in zip(d2s, ms):
                    plsc.addupdate_scatter(hist, [d2], ones, mask=m)
                return c

            for l in range(LANES):
                region(l, whist, 0)
            reduce_hist(256)
            suffix_scan(256 // LANES)
            kneed = _splat(K) - g1
            b2 = count_ge(256 // LANES, kneed) - 1
            g2 = plsc.load_gather(suf, [b2 + 1])
            c2 = plsc.load_gather(suf, [b2]) - g2
            t17 = b1 * 256 + b2
            n2 = g1 + g2 + c2

            base = t17 * (1 << 15)

            def wkeep(idxs, ubs, valids, carry):
                c, himax = carry
                keeps = [v & (_srl(ub, 15) >= t17)
                         for ub, v in zip(ubs, valids)]
                ubks = [ub - base for ub in ubs]
                scs = [plsc.scan_count(zero, mask=k) for k in keeps]
                pops = [plsc.all_reduce_population_count(k) for k in keeps]
                for ubk, ix, k, (cnt, _), pop in zip(ubks, idxs, keeps, scs,
                                                     pops):
                    himax = jnp.maximum(
                        himax, jnp.where(k, _srl(ubk, 24), zero))
                    addr = c + cnt - 1
                    ok = k & (addr < CAP2)
                    plsc.store_scatter(ska, [addr], ubk, mask=ok)
                    plsc.store_scatter(sia, [addr], ix, mask=ok)
                    c = c + pop
                return c, himax

            c0, himax = zero, zero
            for l in range(LANES):
                c0, himax = region(l, wkeep, (c0, himax))
            skip_hi = plsc.all_reduce_population_count(himax == zero)[0] == 16

            @pl.when(r + 1 < rpw)
            def _prefetch():
                pltpu.async_copy(x_hbm.at[row + 1], row_ref, sem)

            n2s = jnp.minimum(n2[0], CAP2)
            trips = lax.div(n2s + LANES - 1, LANES)

            digit_passes = [
                (lambda kv, iv: 255 - (_srl(iv, 4) & 255), 256),
                (lambda kv, iv: 15 - (_srl(iv, 12) & 15), 16),
                (lambda kv, iv: kv & 255, 256),
                (lambda kv, iv: _srl(kv, 8) & 255, 256),
                (lambda kv, iv: _srl(kv, 16) & 255, 256),
                (lambda kv, iv: _srl(kv, 24), 256),
            ]

            trips4 = lax.div(n2s + 4 * LANES - 1, 4 * LANES)
            trips2 = lax.div(n2s + 2 * LANES - 1, 2 * LANES)

            src_k, src_i, dst_k, dst_i = ska, sia, skb, sib
            for pno, (dfn, nbins) in enumerate(digit_passes):
                def hbody(j, _, src_k=src_k, src_i=src_i, dfn=dfn):
                    poss = [(j * 4 + u) * LANES for u in range(4)]
                    valids = [(lane + p) < n2 for p in poss]
                    ds = [dfn(src_k[pl.ds(p, LANES)], src_i[pl.ds(p, LANES)])
                          for p in poss]
                    scs = [plsc.scan_count(d, mask=v)
                           for d, v in zip(ds, valids)]
                    for d, (cnt, last), v in zip(ds, scs, valids):
                        plsc.addupdate_scatter(tot, [d], cnt, mask=last & v)
                    return 0

                def cinit(j, _):
                    cur[pl.ds(j * LANES, LANES)] = plsc.load_gather(
                        suf, [lane + (j * LANES + 1)])
                    return 0

                def perm(j, _, src_k=src_k, src_i=src_i,
                         dst_k=dst_k, dst_i=dst_i, dfn=dfn):
                    poss = [(j * 2 + u) * LANES for u in range(2)]
                    valids = [(lane + p) < n2 for p in poss]
                    kvs = [src_k[pl.ds(p, LANES)] for p in poss]
                    ivs = [src_i[pl.ds(p, LANES)] for p in poss]
                    ds = [dfn(kv, iv) for kv, iv in zip(kvs, ivs)]
                    scs = [plsc.scan_count(d, mask=v)
                           for d, v in zip(ds, valids)]
                    for kv, iv, d, (cnt, last), v in zip(kvs, ivs, ds, scs,
                                                         valids):
                        addr = plsc.load_gather(cur, [d], mask=v) + cnt - 1
                        plsc.store_scatter(dst_k, [addr], kv, mask=v)
                        plsc.store_scatter(dst_i, [addr], iv, mask=v)
                        plsc.addupdate_scatter(cur, [d], cnt, mask=last & v)
                    return 0

                def copy_body(j, _, src_k=src_k, src_i=src_i,
                              dst_k=dst_k, dst_i=dst_i):
                    for u in range(4):
                        sl = pl.ds((j * 4 + u) * LANES, LANES)
                        dst_k[sl] = src_k[sl]
                        dst_i[sl] = src_i[sl]
                    return 0

                if pno == len(digit_passes) - 1:
                    @pl.when(skip_hi)
                    def _copy():
                        lax.fori_loop(0, trips4, copy_body, 0)

                    @pl.when(jnp.logical_not(skip_hi))
                    def _full():
                        lax.fori_loop(0, trips4, hbody, 0)
                        suffix_scan(nbins // LANES)
                        lax.fori_loop(0, nbins // LANES, cinit, 0)
                        lax.fori_loop(0, trips2, perm, 0)
                else:
                    lax.fori_loop(0, trips4, hbody, 0)
                    suffix_scan(nbins // LANES)
                    lax.fori_loop(0, nbins // LANES, cinit, 0)
                    lax.fori_loop(0, trips2, perm, 0)
                src_k, src_i, dst_k, dst_i = dst_k, dst_i, src_k, src_i

            pltpu.sync_copy(src_i.at[pl.ds(0, K)], out_hbm.at[row])
            return b1

        lax.fori_loop(0, rpw, do_row, _splat(HB))

    return topk_idx


def kernel(input_tensor):
    return _make_kernel()(input_tensor)


# speedup vs baseline: 1.0220x; 1.0220x over previous
import functools

import jax
import jax.numpy as jnp
from jax import lax
from jax.experimental import pallas as pl
from jax.experimental.pallas import tpu as pltpu
from jax.experimental.pallas import tpu_sc as plsc

R = 128
L = 32768
K = 1024
LANES = 16
NV = L // LANES
CAPL = 512
CAP2 = 2048
HB = 512


def _srl(x, n):
    return lax.shift_right_logical(x, jnp.full(x.shape, n, jnp.int32))


def _sra(x, n):
    return lax.shift_right_arithmetic(x, jnp.full(x.shape, n, jnp.int32))


def _iota():
    return lax.iota(jnp.int32, LANES)


def _splat(v):
    return jnp.full((LANES,), v, jnp.int32)


def _to_ub(f32v):
    b = lax.bitcast_convert_type(f32v, jnp.int32)
    return b ^ (_sra(b, 31) | _splat(-0x80000000))


def _make_kernel():
    info = plsc.get_sparse_core_info()
    nc, ns = info.num_cores, info.num_subcores
    nw = nc * ns
    rpw = R // nw
    mesh = plsc.VectorSubcoreMesh(core_axis_name="c", subcore_axis_name="s",
                                  num_cores=nc, num_subcores=ns)

    @functools.partial(
        pl.kernel,
        mesh=mesh,
        out_type=jax.ShapeDtypeStruct((R, K), jnp.int32),
        compiler_params=pltpu.CompilerParams(needs_layout_passes=False),
        scratch_types=[
            pltpu.VMEM((L,), jnp.float32),
            pltpu.VMEM((LANES * CAPL,), jnp.int32),
            pltpu.VMEM((CAP2,), jnp.int32),
            pltpu.VMEM((CAP2,), jnp.int32),
            pltpu.VMEM((CAP2,), jnp.int32),
            pltpu.VMEM((CAP2,), jnp.int32),
            pltpu.VMEM((LANES * HB,), jnp.int32),
            pltpu.VMEM((HB,), jnp.int32),
            pltpu.VMEM((HB + LANES,), jnp.int32),
            pltpu.VMEM((272,), jnp.int32),
            pltpu.SemaphoreType.DMA,
        ],
    )
    def topk_idx(x_hbm, out_hbm, row_ref, ci, ska, sia, skb, sib,
                 hist, tot, suf, cur, sem):
        cid = lax.axis_index("c")
        sid = lax.axis_index("s")
        wid = sid * nc + cid
        lane = _iota()
        ones = _splat(1)
        zero = _splat(0)
        lane_hb = lane * HB
        lane_cap = lane * CAPL

        def compact8(cu, ms, offs):
            incs = [m.astype(jnp.int32) for m in ms]
            a01 = incs[0] + incs[1]
            a23 = incs[2] + incs[3]
            a45 = incs[4] + incs[5]
            a0123 = a01 + a23
            total = a0123 + a45 + incs[6] + incs[7]
            o = [zero, incs[0], a01, a01 + incs[2], a0123, a0123 + incs[4],
                 a0123 + a45, a0123 + a45 + incs[6]]
            addrs = [cu + ou for ou in o]
            oks = [m & (a < CAPL) for m, a in zip(ms, addrs)]
            for a, ok, of in zip(addrs, oks, offs):
                plsc.store_scatter(ci, [lane_cap + a], lane + of, mask=ok)
            return cu + total

        def reduce_hist(nbins):
            def body(j, _):
                sls = [pl.ds(l * HB + j * LANES, LANES) for l in range(LANES)]
                vs = [hist[sl] for sl in sls]
                for sl in sls:
                    hist[sl] = zero
                while len(vs) > 1:
                    vs = [a + b for a, b in zip(vs[::2], vs[1::2])]
                tot[pl.ds(j * LANES, LANES)] = vs[0]
                return 0
            lax.fori_loop(0, nbins // LANES, body, 0)

        def reduce_hist_remap():
            def pos(j, _):
                sls = [pl.ds(l * HB + j * LANES, LANES) for l in range(LANES)]
                vs = [hist[sl] for sl in sls]
                for sl in sls:
                    hist[sl] = zero
                while len(vs) > 1:
                    vs = [a + b for a, b in zip(vs[::2], vs[1::2])]
                tot[pl.ds(256 + j * LANES, LANES)] = vs[0]
                return 0
            lax.fori_loop(0, 256 // LANES, pos, 0)

            def neg(j, _):
                sls = [pl.ds(l * HB + 256 + j * LANES, LANES)
                       for l in range(LANES)]
                vs = [hist[sl] for sl in sls]
                for sl in sls:
                    hist[sl] = zero
                while len(vs) > 1:
                    vs = [a + b for a, b in zip(vs[::2], vs[1::2])]
                tot[pl.ds(240 - j * LANES, LANES)] = lax.rev(vs[0], (0,))
                return 0
            lax.fori_loop(0, 256 // LANES, neg, 0)

        def suffix_scan(nchunks):
            suf[pl.ds(nchunks * LANES, LANES)] = zero

            def body(i, carry):
                j = nchunks - 1 - i
                sl = pl.ds(j * LANES, LANES)
                v = tot[sl]
                tot[sl] = zero
                c = lax.rev(plsc.cumsum(lax.rev(v, (0,))), (0,)) + carry
                suf[sl] = c
                return plsc.load_gather(suf, [_splat(0) + j * LANES])

            lax.fori_loop(0, nchunks, body, zero)

        def count_ge(nchunks, kneed):
            def body(j, acc):
                m = suf[pl.ds(j * LANES, LANES)] >= kneed
                return acc + plsc.all_reduce_population_count(m)
            return lax.fori_loop(0, nchunks, body, zero)

        def hclear(j, _):
            hist[pl.ds(j * LANES, LANES)] = zero
            return 0
        lax.fori_loop(0, LANES * HB // LANES, hclear, 0)

        def tclear(j, _):
            tot[pl.ds(j * LANES, LANES)] = zero
            return 0
        lax.fori_loop(0, HB // LANES, tclear, 0)

        pltpu.async_copy(x_hbm.at[wid * rpw], row_ref, sem)

        def do_row(r, bspec):
            row = wid * rpw + r
            pltpu.make_async_copy(x_hbm.at[row], row_ref, sem).wait()

            bok = (bspec >= 256) & (bspec <= 511)
            vthresh = jnp.where(bok, (bspec - 256) * (1 << 23),
                                _splat(0x7FFFF000))

            def p1(i, cu):
                sls = [pl.ds((i * 8 + u) * LANES, LANES) for u in range(8)]
                offs = [(i * 8 + u) * LANES for u in range(8)]
                bs = [lax.bitcast_convert_type(row_ref[sl], jnp.int32)
                      for sl in sls]
                d1s = [_srl(b, 23) for b in bs]
                ms = [b >= vthresh for b in bs]
                for d1 in d1s:
                    plsc.addupdate_scatter(hist, [lane_hb + d1], ones)
                return compact8(cu, ms, offs)

            cuspec = lax.fori_loop(0, NV // 8, p1, zero)
            reduce_hist_remap()
            suffix_scan(HB // LANES)
            b1 = count_ge(HB // LANES, _splat(K)) - 1
            g1 = plsc.load_gather(suf, [b1 + 1])

            fits = plsc.all_reduce_population_count(cuspec <= CAPL)
            hit = ((b1 >= bspec) & (fits == LANES) & bok).astype(jnp.int32)
            cur[pl.ds(0, LANES)] = cuspec

            @pl.when(hit[0] == 0)
            def _p2():
                def p2(i, cu):
                    offs = [(i * 8 + u) * LANES for u in range(8)]
                    bs = [lax.bitcast_convert_type(
                        row_ref[pl.ds(o, LANES)], jnp.int32) for o in offs]
                    ss = [_sra(b, 31) | _splat(-0x80000000) for b in bs]
                    ms = [_srl(b ^ sgn, 23) >= b1 for b, sgn in zip(bs, ss)]
                    return compact8(cu, ms, offs)

                cur[pl.ds(0, LANES)] = lax.fori_loop(0, NV // 8, p2, zero)

            wcnt = jnp.minimum(cur[pl.ds(0, LANES)], CAPL)

            NB = 4

            def region(l, body_fn, carry):
                cl = wcnt[l]
                cls = jnp.full((LANES,), cl, jnp.int32)

                def wrap(j, c):
                    poss = [(j * NB + u) * LANES for u in range(NB)]
                    valids = [(lane + p) < cls for p in poss]
                    idxs = [ci[pl.ds(l * CAPL + p, LANES)] & (L - 1)
                            for p in poss]
                    ubs = [_to_ub(plsc.load_gather(row_ref, [ix], mask=v))
                           for ix, v in zip(idxs, valids)]
                    return body_fn(idxs, ubs, valids, c)

                return lax.fori_loop(
                    0, lax.div(cl + NB * LANES - 1, NB * LANES), wrap, carry)

            def whist(idxs, ubs, valids, c):
                ms = [v & (_srl(ub, 23) == b1) for ub, v in zip(ubs, valids)]
                d2s = [lane_hb + (_srl(ub, 15) & 255) for ub in ubs]
                for d2, m in zip(d2s, ms):
                    plsc.addupdate_scatter(hist, [d2], ones, mask=m)
                return c

            for l in range(LANES):
                region(l, whist, 0)
            reduce_hist(256)
            suffix_scan(256 // LANES)
            kneed = _splat(K) - g1
            b2 = count_ge(256 // LANES, kneed) - 1
            g2 = plsc.load_gather(suf, [b2 + 1])
            c2 = plsc.load_gather(suf, [b2]) - g2
            t17 = b1 * 256 + b2
            n2 = g1 + g2 + c2

            base = t17 * (1 << 15)

            def wkeep(idxs, ubs, valids, carry):
                c, himax = carry
                keeps = [v & (_srl(ub, 15) >= t17)
                         for ub, v in zip(ubs, valids)]
                ubks = [ub - base for ub in ubs]
                scs = [plsc.scan_count(zero, mask=k) for k in keeps]
                pops = [plsc.all_reduce_population_count(k) for k in keeps]
                for ubk, ix, k, (cnt, _), pop in zip(ubks, idxs, keeps, scs,
                                                     pops):
                    himax = jnp.maximum(
                        himax, jnp.where(k, _srl(ubk, 24), zero))
                    addr = c + cnt - 1
                    ok = k & (addr < CAP2)
                    plsc.store_scatter(ska, [addr], ubk, mask=ok)
                    plsc.store_scatter(sia, [addr], ix, mask=ok)
                    c = c + pop
                return c, himax

            c0, himax = zero, zero
            for l in range(LANES):
                c0, himax = region(l, wkeep, (c0, himax))
            skip_hi = plsc.all_reduce_population_count(himax == zero)[0] == 16

            @pl.when(r + 1 < rpw)
            def _prefetch():
                pltpu.async_copy(x_hbm.at[row + 1], row_ref, sem)

            n2s = jnp.minimum(n2[0], CAP2)
            trips = lax.div(n2s + LANES - 1, LANES)

            digit_passes = [
                (lambda kv, iv: 255 - (_srl(iv, 4) & 255), 256),
                (lambda kv, iv: 15 - (_srl(iv, 12) & 15), 16),
                (lambda kv, iv: kv & 255, 256),
                (lambda kv, iv: _srl(kv, 8) & 255, 256),
                (lambda kv, iv: _srl(kv, 16) & 255, 256),
                (lambda kv, iv: _srl(kv, 24), 256),
            ]

            trips4 = lax.div(n2s + 4 * LANES - 1, 4 * LANES)
            trips2 = lax.div(n2s + 2 * LANES - 1, 2 * LANES)

            src_k, src_i, dst_k, dst_i = ska, sia, skb, sib
            for pno, (dfn, nbins) in enumerate(digit_passes):
                def hbody(j, _, src_k=src_k, src_i=src_i, dfn=dfn):
                    poss = [(j * 4 + u) * LANES for u in range(4)]
                    valids = [(lane + p) < n2 for p in poss]
                    ds = [dfn(src_k[pl.ds(p, LANES)], src_i[pl.ds(p, LANES)])
                          for p in poss]
                    scs = [plsc.scan_count(d, mask=v)
                           for d, v in zip(ds, valids)]
                    for d, (cnt, last), v in zip(ds, scs, valids):
                        plsc.addupdate_scatter(tot, [d], cnt, mask=last & v)
                    return 0

                def cinit(j, _):
                    cur[pl.ds(j * LANES, LANES)] = plsc.load_gather(
                        suf, [lane + (j * LANES + 1)])
                    return 0

                def perm(j, _, src_k=src_k, src_i=src_i,
                         dst_k=dst_k, dst_i=dst_i, dfn=dfn):
                    poss = [(j * 2 + u) * LANES for u in range(2)]
                    valids = [(lane + p) < n2 for p in poss]
                    kvs = [src_k[pl.ds(p, LANES)] for p in poss]
                    ivs = [src_i[pl.ds(p, LANES)] for p in poss]
                    ds = [dfn(kv, iv) for kv, iv in zip(kvs, ivs)]
                    scs = [plsc.scan_count(d, mask=v)
                           for d, v in zip(ds, valids)]
                    for kv, iv, d, (cnt, last), v in zip(kvs, ivs, ds, scs,
                                                         valids):
                        addr = plsc.load_gather(cur, [d], mask=v) + cnt - 1
                        plsc.store_scatter(dst_k, [addr], kv, mask=v)
                        plsc.store_scatter(dst_i, [addr], iv, mask=v)
                        plsc.addupdate_scatter(cur, [d], cnt, mask=last & v)
                    return 0

                def copy_body(j, _, src_k=src_k, src_i=src_i,
                              dst_k=dst_k, dst_i=dst_i):
                    for u in range(4):
                        sl = pl.ds((j * 4 + u) * LANES, LANES)
                        dst_k[sl] = src_k[sl]
                        dst_i[sl] = src_i[sl]
                    return 0

                if pno == len(digit_passes) - 1:
                    @pl.when(skip_hi)
                    def _copy():
                        lax.fori_loop(0, trips4, copy_body, 0)

                    @pl.when(jnp.logical_not(skip_hi))
                    def _full():
                        lax.fori_loop(0, trips4, hbody, 0)
                        suffix_scan(nbins // LANES)
                        lax.fori_loop(0, nbins // LANES, cinit, 0)
                        lax.fori_loop(0, trips2, perm, 0)
                else:
                    lax.fori_loop(0, trips4, hbody, 0)
                    suffix_scan(nbins // LANES)
                    lax.fori_loop(0, nbins // LANES, cinit, 0)
                    lax.fori_loop(0, trips2, perm, 0)
                src_k, src_i, dst_k, dst_i = dst_k, dst_i, src_k, src_i

            pltpu.sync_copy(src_i.at[pl.ds(0, K)], out_hbm.at[row])
            return b1

        lax.fori_loop(0, rpw, do_row, _splat(HB))

    return topk_idx


def kernel(input_tensor):
    return _make_kernel()(input_tensor)


# speedup vs baseline: 1.0399x; 1.0176x over previous
import functools

import jax
import jax.numpy as jnp
from jax import lax
from jax.experimental import pallas as pl
from jax.experimental.pallas import tpu as pltpu
from jax.experimental.pallas import tpu_sc as plsc

R = 128
L = 32768
K = 1024
LANES = 16
NV = L // LANES
CAPL = 512
CAP2 = 2048
HB = 512


def _srl(x, n):
    return lax.shift_right_logical(x, jnp.full(x.shape, n, jnp.int32))


def _sra(x, n):
    return lax.shift_right_arithmetic(x, jnp.full(x.shape, n, jnp.int32))


def _iota():
    return lax.iota(jnp.int32, LANES)


def _splat(v):
    return jnp.full((LANES,), v, jnp.int32)


def _to_ub(f32v):
    b = lax.bitcast_convert_type(f32v, jnp.int32)
    return b ^ (_sra(b, 31) | _splat(-0x80000000))


def _make_kernel():
    info = plsc.get_sparse_core_info()
    nc, ns = info.num_cores, info.num_subcores
    nw = nc * ns
    rpw = R // nw
    mesh = plsc.VectorSubcoreMesh(core_axis_name="c", subcore_axis_name="s",
                                  num_cores=nc, num_subcores=ns)

    @functools.partial(
        pl.kernel,
        mesh=mesh,
        out_type=jax.ShapeDtypeStruct((R, K), jnp.int32),
        compiler_params=pltpu.CompilerParams(needs_layout_passes=False),
        scratch_types=[
            pltpu.VMEM((L,), jnp.float32),
            pltpu.VMEM((LANES * CAPL,), jnp.int32),
            pltpu.VMEM((CAP2,), jnp.int32),
            pltpu.VMEM((CAP2,), jnp.int32),
            pltpu.VMEM((CAP2,), jnp.int32),
            pltpu.VMEM((CAP2,), jnp.int32),
            pltpu.VMEM((LANES * HB,), jnp.int32),
            pltpu.VMEM((HB,), jnp.int32),
            pltpu.VMEM((HB + LANES,), jnp.int32),
            pltpu.VMEM((272,), jnp.int32),
            pltpu.SemaphoreType.DMA,
        ],
    )
    def topk_idx(x_hbm, out_hbm, row_ref, ci, ska, sia, skb, sib,
                 hist, tot, suf, cur, sem):
        cid = lax.axis_index("c")
        sid = lax.axis_index("s")
        wid = sid * nc + cid
        lane = _iota()
        ones = _splat(1)
        zero = _splat(0)
        lane_hb = lane * HB
        lane_cap = lane * CAPL

        def compact8(cu, ms, offs):
            incs = [m.astype(jnp.int32) for m in ms]
            a01 = incs[0] + incs[1]
            a23 = incs[2] + incs[3]
            a45 = incs[4] + incs[5]
            a0123 = a01 + a23
            total = a0123 + a45 + incs[6] + incs[7]
            o = [zero, incs[0], a01, a01 + incs[2], a0123, a0123 + incs[4],
                 a0123 + a45, a0123 + a45 + incs[6]]
            addrs = [cu + ou for ou in o]
            oks = [m & (a < CAPL) for m, a in zip(ms, addrs)]
            for a, ok, of in zip(addrs, oks, offs):
                plsc.store_scatter(ci, [lane_cap + a], lane + of, mask=ok)
            return cu + total

        def reduce_hist(nbins):
            def body(j, _):
                sls = [pl.ds(l * HB + j * LANES, LANES) for l in range(LANES)]
                vs = [hist[sl] for sl in sls]
                for sl in sls:
                    hist[sl] = zero
                while len(vs) > 1:
                    vs = [a + b for a, b in zip(vs[::2], vs[1::2])]
                tot[pl.ds(j * LANES, LANES)] = vs[0]
                return 0
            lax.fori_loop(0, nbins // LANES, body, 0)

        def suffix_scan(nchunks):
            suf[pl.ds(nchunks * LANES, LANES)] = zero

            def body(i, carry):
                j = nchunks - 1 - i
                sl = pl.ds(j * LANES, LANES)
                v = tot[sl]
                tot[sl] = zero
                c = lax.rev(plsc.cumsum(lax.rev(v, (0,))), (0,)) + carry
                suf[sl] = c
                return plsc.load_gather(suf, [_splat(0) + j * LANES])

            lax.fori_loop(0, nchunks, body, zero)

        def count_ge(nchunks, kneed):
            def body(j, acc):
                m = suf[pl.ds(j * LANES, LANES)] >= kneed
                return acc + plsc.all_reduce_population_count(m)
            return lax.fori_loop(0, nchunks, body, zero)

        def hclear(j, _):
            hist[pl.ds(j * LANES, LANES)] = zero
            return 0
        lax.fori_loop(0, LANES * HB // LANES, hclear, 0)

        def tclear(j, _):
            tot[pl.ds(j * LANES, LANES)] = zero
            return 0
        lax.fori_loop(0, HB // LANES, tclear, 0)

        pltpu.async_copy(x_hbm.at[wid * rpw], row_ref, sem)

        def do_row(r, bspec):
            row = wid * rpw + r
            pltpu.make_async_copy(x_hbm.at[row], row_ref, sem).wait()

            def p1(i, cu):
                sls = [pl.ds((i * 8 + u) * LANES, LANES) for u in range(8)]
                offs = [(i * 8 + u) * LANES for u in range(8)]
                bs = [lax.bitcast_convert_type(row_ref[sl], jnp.int32)
                      for sl in sls]
                ss = [_sra(b, 31) | _splat(-0x80000000) for b in bs]
                d1s = [_srl(b ^ sgn, 23) for b, sgn in zip(bs, ss)]
                ms = [d1 >= bspec for d1 in d1s]
                for d1 in d1s:
                    plsc.addupdate_scatter(hist, [lane_hb + d1], ones)
                return compact8(cu, ms, offs)

            cuspec = lax.fori_loop(0, NV // 8, p1, zero)
            reduce_hist(HB)
            suffix_scan(HB // LANES)
            b1 = count_ge(HB // LANES, _splat(K)) - 1
            g1 = plsc.load_gather(suf, [b1 + 1])

            fits = plsc.all_reduce_population_count(cuspec <= CAPL)
            hit = ((b1 >= bspec) & (fits == LANES)).astype(jnp.int32)
            cur[pl.ds(0, LANES)] = cuspec

            @pl.when(hit[0] == 0)
            def _p2():
                def p2(i, cu):
                    offs = [(i * 8 + u) * LANES for u in range(8)]
                    bs = [lax.bitcast_convert_type(
                        row_ref[pl.ds(o, LANES)], jnp.int32) for o in offs]
                    ss = [_sra(b, 31) | _splat(-0x80000000) for b in bs]
                    ms = [_srl(b ^ sgn, 23) >= b1 for b, sgn in zip(bs, ss)]
                    return compact8(cu, ms, offs)

                cur[pl.ds(0, LANES)] = lax.fori_loop(0, NV // 8, p2, zero)

            wcnt = jnp.minimum(cur[pl.ds(0, LANES)], CAPL)

            NB = 4

            def region(l, body_fn, carry):
                cl = wcnt[l]
                cls = jnp.full((LANES,), cl, jnp.int32)

                def wrap(j, c):
                    poss = [(j * NB + u) * LANES for u in range(NB)]
                    valids = [(lane + p) < cls for p in poss]
                    idxs = [ci[pl.ds(l * CAPL + p, LANES)] & (L - 1)
                            for p in poss]
                    ubs = [_to_ub(plsc.load_gather(row_ref, [ix], mask=v))
                           for ix, v in zip(idxs, valids)]
                    return body_fn(idxs, ubs, valids, c)

                return lax.fori_loop(
                    0, lax.div(cl + NB * LANES - 1, NB * LANES), wrap, carry)

            def whist(idxs, ubs, valids, c):
                ms = [v & (_srl(ub, 23) == b1) for ub, v in zip(ubs, valids)]
                d2s = [lane_hb + (_srl(ub, 15) & 255) for ub in ubs]
                for d2, m in zip(d2s, ms):
                    plsc.addupdate_scatter(hist, [d2], ones, mask=m)
                return c

            for l in range(LANES):
                region(l, whist, 0)
            reduce_hist(256)
            suffix_scan(256 // LANES)
            kneed = _splat(K) - g1
            b2 = count_ge(256 // LANES, kneed) - 1
            g2 = plsc.load_gather(suf, [b2 + 1])
            c2 = plsc.load_gather(suf, [b2]) - g2
            t17 = b1 * 256 + b2
            n2 = g1 + g2 + c2

            base = t17 * (1 << 15)

            def wkeep(idxs, ubs, valids, carry):
                c, himax = carry
                keeps = [v & (_srl(ub, 15) >= t17)
                         for ub, v in zip(ubs, valids)]
                ubks = [ub - base for ub in ubs]
                scs = [plsc.scan_count(zero, mask=k) for k in keeps]
                pops = [plsc.all_reduce_population_count(k) for k in keeps]
                for ubk, ix, k, (cnt, _), pop in zip(ubks, idxs, keeps, scs,
                                                     pops):
                    himax = jnp.maximum(
                        himax, jnp.where(k, _srl(ubk, 24), zero))
                    addr = c + cnt - 1
                    ok = k & (addr < CAP2)
                    plsc.store_scatter(ska, [addr], ubk, mask=ok)
                    plsc.store_scatter(sia, [addr], ix, mask=ok)
                    c = c + pop
                return c, himax

            c0, himax = zero, zero
            for l in range(LANES):
                c0, himax = region(l, wkeep, (c0, himax))
            skip_hi = plsc.all_reduce_population_count(himax == zero)[0] == 16

            @pl.when(r + 1 < rpw)
            def _prefetch():
                pltpu.async_copy(x_hbm.at[row + 1], row_ref, sem)

            n2s = jnp.minimum(n2[0], CAP2)
            trips = lax.div(n2s + LANES - 1, LANES)

            digit_passes = [
                (lambda kv, iv: 255 - (_srl(iv, 4) & 255), 256),
                (lambda kv, iv: 15 - (_srl(iv, 12) & 15), 16),
                (lambda kv, iv: kv & 255, 256),
                (lambda kv, iv: _srl(kv, 8) & 255, 256),
                (lambda kv, iv: _srl(kv, 16) & 255, 256),
                (lambda kv, iv: _srl(kv, 24), 256),
            ]

            trips4 = lax.div(n2s + 4 * LANES - 1, 4 * LANES)
            trips2 = lax.div(n2s + 2 * LANES - 1, 2 * LANES)

            src_k, src_i, dst_k, dst_i = ska, sia, skb, sib
            for pno, (dfn, nbins) in enumerate(digit_passes):
                def hbody(j, _, src_k=src_k, src_i=src_i, dfn=dfn):
                    poss = [(j * 4 + u) * LANES for u in range(4)]
                    valids = [(lane + p) < n2 for p in poss]
                    ds = [dfn(src_k[pl.ds(p, LANES)], src_i[pl.ds(p, LANES)])
                          for p in poss]
                    scs = [plsc.scan_count(d, mask=v)
                           for d, v in zip(ds, valids)]
                    for d, (cnt, last), v in zip(ds, scs, valids):
                        plsc.addupdate_scatter(tot, [d], cnt, mask=last & v)
                    return 0

                def cinit(j, _):
                    cur[pl.ds(j * LANES, LANES)] = plsc.load_gather(
                        suf, [lane + (j * LANES + 1)])
                    return 0

                def perm(j, _, src_k=src_k, src_i=src_i,
                         dst_k=dst_k, dst_i=dst_i, dfn=dfn):
                    poss = [(j * 2 + u) * LANES for u in range(2)]
                    valids = [(lane + p) < n2 for p in poss]
                    kvs = [src_k[pl.ds(p, LANES)] for p in poss]
                    ivs = [src_i[pl.ds(p, LANES)] for p in poss]
                    ds = [dfn(kv, iv) for kv, iv in zip(kvs, ivs)]
                    scs = [plsc.scan_count(d, mask=v)
                           for d, v in zip(ds, valids)]
                    for kv, iv, d, (cnt, last), v in zip(kvs, ivs, ds, scs,
                                                         valids):
                        addr = plsc.load_gather(cur, [d], mask=v) + cnt - 1
                        plsc.store_scatter(dst_k, [addr], kv, mask=v)
                        plsc.store_scatter(dst_i, [addr], iv, mask=v)
                        plsc.addupdate_scatter(cur, [d], cnt, mask=last & v)
                    return 0

                def copy_body(j, _, src_k=src_k, src_i=src_i,
                              dst_k=dst_k, dst_i=dst_i):
                    for u in range(4):
                        sl = pl.ds((j * 4 + u) * LANES, LANES)
                        dst_k[sl] = src_k[sl]
                        dst_i[sl] = src_i[sl]
                    return 0

                if pno == len(digit_passes) - 1:
                    @pl.when(skip_hi)
                    def _copy():
                        lax.fori_loop(0, trips4, copy_body, 0)

                    @pl.when(jnp.logical_not(skip_hi))
                    def _full():
                        lax.fori_loop(0, trips4, hbody, 0)
                        suffix_scan(nbins // LANES)
                        lax.fori_loop(0, nbins // LANES, cinit, 0)
                        lax.fori_loop(0, trips2, perm, 0)
                else:
                    lax.fori_loop(0, trips4, hbody, 0)
                    suffix_scan(nbins // LANES)
                    lax.fori_loop(0, nbins // LANES, cinit, 0)
                    lax.fori_loop(0, trips2, perm, 0)
                src_k, src_i, dst_k, dst_i = dst_k, dst_i, src_k, src_i

            pltpu.sync_copy(src_i.at[pl.ds(0, K)], out_hbm.at[row])
            return b1

        lax.fori_loop(0, rpw, do_row, _splat(HB))

    return topk_idx


def kernel(input_tensor):
    return _make_kernel()(input_tensor)
